# baseline (speedup 1.0000x reference)
import jax
import jax.numpy as jnp
from jax import lax
from jax.experimental import pallas as pl
from jax.experimental.pallas import tpu as pltpu

NUM_HEADS = 128
QK_NOPE = 128
QK_ROPE = 64
KV_LORA = 512
V_DIM = 128
TOPK = 1024
IDX_HEADS = 8
IDX_DIM = 64
B = 64
KV = 2048
SOFTMAX_SCALE = (KV_LORA + QK_ROPE) ** (-0.5)
NEG = -1e30
HCHUNK = 8
NSPLIT = 4
_ROWS_Q = KV // NSPLIT


def _scores_kernel(qr_ref, ik_ref, s_ref):
    qr = qr_ref[0]
    ik = ik_ref[0]
    s8 = lax.dot_general(qr, ik, (((1,), (1,)), ((), ())),
                         preferred_element_type=jnp.float32)
    s_ref[0] = jnp.sum(jnp.maximum(s8, 0.0), axis=0, keepdims=True)


def _thresh_kernel(s_ref, bt_ref):
    s = s_ref[:, 0, :]
    si = lax.bitcast_convert_type(s, jnp.int32)

    def body(_, carry):
        lo, hi = carry
        mid = lo + ((hi - lo) >> 1)
        ge = (si >= mid).astype(jnp.float32)
        cnt = jnp.sum(ge, axis=1, keepdims=True)
        pred = cnt >= TOPK
        return jnp.where(pred, mid, lo), jnp.where(pred, hi, mid)

    lo0 = jnp.zeros((B, 1), jnp.int32)
    hi0 = jnp.full((B, 1), 0x7F800000, jnp.int32)
    lo, _ = lax.fori_loop(0, 31, body, (lo0, hi0))
    m01 = (si >= lo).astype(jnp.float32)
    eye = (lax.broadcasted_iota(jnp.int32, (B, B), 0)
           == lax.broadcasted_iota(jnp.int32, (B, B), 1)).astype(jnp.float32)
    m01t = lax.dot_general(m01, eye, (((0,), (0,)), ((), ())),
                           preferred_element_type=jnp.float32)
    bt_ref[...] = NEG * (1.0 - m01t)


def _qabsorb_kernel(qn_ref, kbt_ref, o_ref):
    for i in range(HCHUNK):
        qn = qn_ref[:, i, :].astype(jnp.bfloat16)
        kbt = kbt_ref[i].astype(jnp.bfloat16)
        o_ref[:, i, :] = (SOFTMAX_SCALE * lax.dot_general(
            qn, kbt, (((1,), (1,)), ((), ())),
            preferred_element_type=jnp.float32)).astype(jnp.bfloat16)


def _attn_kernel(qno_ref, qr_ref, kv_hbm, bt_ref, o_ref,
                 buf0, buf1, sems0, sems1):
    b = pl.program_id(0)

    def issue(buf, sems, bb):
        for qi in range(NSPLIT):
            pltpu.make_async_copy(
                kv_hbm.at[bb, pl.ds(qi * _ROWS_Q, _ROWS_Q), :],
                buf.at[pl.ds(qi * _ROWS_Q, _ROWS_Q), :],
                sems.at[qi],
            ).start()

    def wait(buf, sems, bb):
        for qi in range(NSPLIT):
            pltpu.make_async_copy(
                kv_hbm.at[bb, pl.ds(qi * _ROWS_Q, _ROWS_Q), :],
                buf.at[pl.ds(qi * _ROWS_Q, _ROWS_Q), :],
                sems.at[qi],
            ).wait()

    @pl.when(b == 0)
    def _():
        issue(buf0, sems0, 0)

    @pl.when((b % 2 == 0) & (b + 1 < B))
    def _():
        issue(buf1, sems1, b + 1)

    @pl.when((b % 2 == 1) & (b + 1 < B))
    def _():
        issue(buf0, sems0, b + 1)

    def compute(buf, sems):
        wait(buf, sems, b)
        qno = qno_ref[0]
        qrope = (qr_ref[0] * SOFTMAX_SCALE).astype(jnp.bfloat16)
        kv = buf[...].astype(jnp.bfloat16)
        onehot = (lax.broadcasted_iota(jnp.int32, (1, B), 1)
                  == b).astype(jnp.float32)
        bias = lax.dot_general(
            bt_ref[...], onehot, (((1,), (1,)), ((), ())),
            preferred_element_type=jnp.float32)
        lt = lax.dot_general(
            kv[:, :KV_LORA], qno, (((1,), (1,)), ((), ())),
            preferred_element_type=jnp.float32)
        lt += lax.dot_general(
            kv[:, KV_LORA:], qrope, (((1,), (1,)), ((), ())),
            preferred_element_type=jnp.float32)
        lt += bias
        m = jnp.max(lt, axis=0, keepdims=True)
        p = jnp.exp(lt - m)
        attn_t = (p / jnp.sum(p, axis=0, keepdims=True)).astype(jnp.bfloat16)
        o_ref[0] = lax.dot_general(
            attn_t, kv[:, :KV_LORA], (((0,), (0,)), ((), ())),
            preferred_element_type=jnp.float32).astype(jnp.bfloat16)

    @pl.when(b % 2 == 0)
    def _():
        compute(buf0, sems0)

    @pl.when(b % 2 == 1)
    def _():
        compute(buf1, sems1)


def _oproj_kernel(ao_ref, vb_ref, o_ref):
    for i in range(HCHUNK):
        ao = ao_ref[:, i, :]
        vb = vb_ref[i].astype(jnp.bfloat16)
        o_ref[:, i, :] = lax.dot_general(
            ao, vb, (((1,), (1,)), ((), ())),
            preferred_element_type=jnp.float32)


@jax.jit
def kernel(qr, q, indexer_k, latent_cache, k_b_proj_trans, v_b_proj):
    scores = pl.pallas_call(
        _scores_kernel,
        grid=(B,),
        in_specs=[
            pl.BlockSpec((1, IDX_HEADS, IDX_DIM), lambda b: (b, 0, 0)),
            pl.BlockSpec((1, KV, IDX_DIM), lambda b: (b, 0, 0)),
        ],
        out_specs=pl.BlockSpec((1, 1, KV), lambda b: (b, 0, 0)),
        out_shape=jax.ShapeDtypeStruct((B, 1, KV), jnp.float32),
    )(qr, indexer_k)

    bias_t = pl.pallas_call(
        _thresh_kernel,
        out_shape=jax.ShapeDtypeStruct((KV, B), jnp.float32),
    )(scores)

    q_nope = q[..., :QK_NOPE]
    q_rope = q[..., QK_NOPE:]

    qno = pl.pallas_call(
        _qabsorb_kernel,
        grid=(NUM_HEADS // HCHUNK,),
        in_specs=[
            pl.BlockSpec((B, HCHUNK, QK_NOPE), lambda h: (0, h, 0)),
            pl.BlockSpec((HCHUNK, KV_LORA, QK_NOPE), lambda h: (h, 0, 0)),
        ],
        out_specs=pl.BlockSpec((B, HCHUNK, KV_LORA), lambda h: (0, h, 0)),
        out_shape=jax.ShapeDtypeStruct((B, NUM_HEADS, KV_LORA), jnp.bfloat16),
    )(q_nope, k_b_proj_trans)

    ao = pl.pallas_call(
        _attn_kernel,
        grid=(B,),
        in_specs=[
            pl.BlockSpec((1, NUM_HEADS, KV_LORA), lambda b: (b, 0, 0)),
            pl.BlockSpec((1, NUM_HEADS, QK_ROPE), lambda b: (b, 0, 0)),
            pl.BlockSpec(memory_space=pl.ANY),
            pl.BlockSpec((KV, B), lambda b: (0, 0)),
        ],
        out_specs=pl.BlockSpec((1, NUM_HEADS, KV_LORA), lambda b: (b, 0, 0)),
        out_shape=jax.ShapeDtypeStruct((B, NUM_HEADS, KV_LORA), jnp.bfloat16),
        scratch_shapes=[
            pltpu.VMEM((KV, KV_LORA + QK_ROPE), jnp.float32),
            pltpu.VMEM((KV, KV_LORA + QK_ROPE), jnp.float32),
            pltpu.SemaphoreType.DMA((NSPLIT,)),
            pltpu.SemaphoreType.DMA((NSPLIT,)),
        ],
    )(qno, q_rope, latent_cache, bias_t)

    out = pl.pallas_call(
        _oproj_kernel,
        grid=(NUM_HEADS // HCHUNK,),
        in_specs=[
            pl.BlockSpec((B, HCHUNK, KV_LORA), lambda h: (0, h, 0)),
            pl.BlockSpec((HCHUNK, V_DIM, KV_LORA), lambda h: (h, 0, 0)),
        ],
        out_specs=pl.BlockSpec((B, HCHUNK, V_DIM), lambda h: (0, h, 0)),
        out_shape=jax.ShapeDtypeStruct((B, NUM_HEADS, V_DIM), jnp.float32),
    )(ao, v_b_proj)

    return out.reshape(B, NUM_HEADS * V_DIM)


# speedup vs baseline: 1.2916x; 1.2916x over previous
import jax
import jax.numpy as jnp
from jax import lax
from jax.experimental import pallas as pl
from jax.experimental.pallas import tpu as pltpu

NUM_HEADS = 128
QK_NOPE = 128
QK_ROPE = 64
KV_LORA = 512
V_DIM = 128
TOPK = 1024
IDX_HEADS = 8
IDX_DIM = 64
B = 64
KV = 2048
SOFTMAX_SCALE = (KV_LORA + QK_ROPE) ** (-0.5)
NEG = -1e30
HCHUNK = 8
NSPLIT = 4
_ROWS_Q = 2 * KV // NSPLIT


def _scores_kernel(qr_ref, ik_ref, s_ref):
    qr = qr_ref[0]
    ik = ik_ref[0]
    s8 = lax.dot_general(qr, ik, (((1,), (1,)), ((), ())),
                         preferred_element_type=jnp.float32)
    s_ref[0] = jnp.sum(jnp.maximum(s8, 0.0), axis=0, keepdims=True)


def _thresh_kernel(s_ref, bt_ref):
    s = s_ref[:, 0, :]
    si = lax.bitcast_convert_type(s, jnp.int32)

    def body(_, carry):
        lo, hi = carry
        mid = lo + ((hi - lo) >> 1)
        ge = (si >= mid).astype(jnp.float32)
        cnt = jnp.sum(ge, axis=1, keepdims=True)
        pred = cnt >= TOPK
        return jnp.where(pred, mid, lo), jnp.where(pred, hi, mid)

    lo0 = jnp.zeros((B, 1), jnp.int32)
    hi0 = jnp.full((B, 1), 0x7F800000, jnp.int32)
    lo, _ = lax.fori_loop(0, 31, body, (lo0, hi0))
    m01 = (si >= lo).astype(jnp.float32)
    eye = (lax.broadcasted_iota(jnp.int32, (B, B), 0)
           == lax.broadcasted_iota(jnp.int32, (B, B), 1)).astype(jnp.float32)
    m01t = lax.dot_general(m01, eye, (((0,), (0,)), ((), ())),
                           preferred_element_type=jnp.float32)
    bt_ref[...] = NEG * (1.0 - m01t)


def _qabsorb_kernel(qn_ref, kbt_ref, o_ref):
    for i in range(HCHUNK):
        qn = qn_ref[:, i, :].astype(jnp.bfloat16)
        kbt = kbt_ref[i].astype(jnp.bfloat16)
        o_ref[:, i, :] = (SOFTMAX_SCALE * lax.dot_general(
            qn, kbt, (((1,), (1,)), ((), ())),
            preferred_element_type=jnp.float32)).astype(jnp.bfloat16)


def _attn_kernel(qno_ref, qr_ref, kv_hbm, bt_ref, o_ref,
                 buf0, buf1, sems0, sems1):
    b = pl.program_id(0)

    def issue(buf, sems, bb):
        for qi in range(NSPLIT):
            pltpu.make_async_copy(
                kv_hbm.at[pl.ds(bb * 2 * KV + qi * _ROWS_Q, _ROWS_Q), :],
                buf.at[pl.ds(qi * _ROWS_Q, _ROWS_Q), :],
                sems.at[qi],
            ).start()

    def wait(buf, sems, bb):
        for qi in range(NSPLIT):
            pltpu.make_async_copy(
                kv_hbm.at[pl.ds(bb * 2 * KV + qi * _ROWS_Q, _ROWS_Q), :],
                buf.at[pl.ds(qi * _ROWS_Q, _ROWS_Q), :],
                sems.at[qi],
            ).wait()

    @pl.when(b == 0)
    def _():
        issue(buf0, sems0, 0)

    @pl.when((b % 2 == 0) & (b + 1 < B // 2))
    def _():
        issue(buf1, sems1, b + 1)

    @pl.when((b % 2 == 1) & (b + 1 < B // 2))
    def _():
        issue(buf0, sems0, b + 1)

    def compute(buf, sems):
        wait(buf, sems, b)
        for j in range(2):
            qno = qno_ref[j]
            qrope = (qr_ref[j] * SOFTMAX_SCALE).astype(jnp.bfloat16)
            kv = buf[pl.ds(j * KV, KV), :].astype(jnp.bfloat16)
            onehot = (lax.broadcasted_iota(jnp.int32, (1, B), 1)
                      == 2 * b + j).astype(jnp.float32)
            bias = lax.dot_general(
                bt_ref[...], onehot, (((1,), (1,)), ((), ())),
                preferred_element_type=jnp.float32)
            lt = lax.dot_general(
                kv[:, :KV_LORA], qno, (((1,), (1,)), ((), ())),
                preferred_element_type=jnp.float32)
            lt += lax.dot_general(
                kv[:, KV_LORA:], qrope, (((1,), (1,)), ((), ())),
                preferred_element_type=jnp.float32)
            lt += bias
            m = jnp.max(lt, axis=0, keepdims=True)
            p = jnp.exp(lt - m)
            attn_t = (p / jnp.sum(p, axis=0,
                                  keepdims=True)).astype(jnp.bfloat16)
            o_ref[j] = lax.dot_general(
                attn_t, kv[:, :KV_LORA], (((0,), (0,)), ((), ())),
                preferred_element_type=jnp.float32).astype(jnp.bfloat16)

    @pl.when(b % 2 == 0)
    def _():
        compute(buf0, sems0)

    @pl.when(b % 2 == 1)
    def _():
        compute(buf1, sems1)


def _oproj_kernel(ao_ref, vb_ref, o_ref):
    for i in range(HCHUNK):
        ao = ao_ref[:, i, :]
        vb = vb_ref[i].astype(jnp.bfloat16)
        o_ref[:, i, :] = lax.dot_general(
            ao, vb, (((1,), (1,)), ((), ())),
            preferred_element_type=jnp.float32)


@jax.jit
def kernel(qr, q, indexer_k, latent_cache, k_b_proj_trans, v_b_proj):
    scores = pl.pallas_call(
        _scores_kernel,
        grid=(B,),
        in_specs=[
            pl.BlockSpec((1, IDX_HEADS, IDX_DIM), lambda b: (b, 0, 0)),
            pl.BlockSpec((1, KV, IDX_DIM), lambda b: (b, 0, 0)),
        ],
        out_specs=pl.BlockSpec((1, 1, KV), lambda b: (b, 0, 0)),
        out_shape=jax.ShapeDtypeStruct((B, 1, KV), jnp.float32),
    )(qr, indexer_k)

    bias_t = pl.pallas_call(
        _thresh_kernel,
        out_shape=jax.ShapeDtypeStruct((KV, B), jnp.float32),
    )(scores)

    q_nope = q[..., :QK_NOPE]
    q_rope = q[..., QK_NOPE:]

    qno = pl.pallas_call(
        _qabsorb_kernel,
        grid=(NUM_HEADS // HCHUNK,),
        in_specs=[
            pl.BlockSpec((B, HCHUNK, QK_NOPE), lambda h: (0, h, 0)),
            pl.BlockSpec((HCHUNK, KV_LORA, QK_NOPE), lambda h: (h, 0, 0)),
        ],
        out_specs=pl.BlockSpec((B, HCHUNK, KV_LORA), lambda h: (0, h, 0)),
        out_shape=jax.ShapeDtypeStruct((B, NUM_HEADS, KV_LORA), jnp.bfloat16),
    )(q_nope, k_b_proj_trans)

    ao = pl.pallas_call(
        _attn_kernel,
        grid=(B // 2,),
        in_specs=[
            pl.BlockSpec((2, NUM_HEADS, KV_LORA), lambda b: (b, 0, 0)),
            pl.BlockSpec((2, NUM_HEADS, QK_ROPE), lambda b: (b, 0, 0)),
            pl.BlockSpec(memory_space=pl.ANY),
            pl.BlockSpec((KV, B), lambda b: (0, 0)),
        ],
        out_specs=pl.BlockSpec((2, NUM_HEADS, KV_LORA), lambda b: (b, 0, 0)),
        out_shape=jax.ShapeDtypeStruct((B, NUM_HEADS, KV_LORA), jnp.bfloat16),
        scratch_shapes=[
            pltpu.VMEM((2 * KV, KV_LORA + QK_ROPE), jnp.float32),
            pltpu.VMEM((2 * KV, KV_LORA + QK_ROPE), jnp.float32),
            pltpu.SemaphoreType.DMA((NSPLIT,)),
            pltpu.SemaphoreType.DMA((NSPLIT,)),
        ],
    )(qno, q_rope,
      latent_cache.reshape(B * KV, KV_LORA + QK_ROPE), bias_t)

    out = pl.pallas_call(
        _oproj_kernel,
        grid=(NUM_HEADS // HCHUNK,),
        in_specs=[
            pl.BlockSpec((B, HCHUNK, KV_LORA), lambda h: (0, h, 0)),
            pl.BlockSpec((HCHUNK, V_DIM, KV_LORA), lambda h: (h, 0, 0)),
        ],
        out_specs=pl.BlockSpec((B, HCHUNK, V_DIM), lambda h: (0, h, 0)),
        out_shape=jax.ShapeDtypeStruct((B, NUM_HEADS, V_DIM), jnp.float32),
    )(ao, v_b_proj)

    return out.reshape(B, NUM_HEADS * V_DIM)


# speedup vs baseline: 1.3081x; 1.0128x over previous
import jax
import jax.numpy as jnp
from jax import lax
from jax.experimental import pallas as pl
from jax.experimental.pallas import tpu as pltpu

NUM_HEADS = 128
QK_NOPE = 128
QK_ROPE = 64
KV_LORA = 512
V_DIM = 128
TOPK = 1024
IDX_HEADS = 8
IDX_DIM = 64
B = 64
KV = 2048
SOFTMAX_SCALE = (KV_LORA + QK_ROPE) ** (-0.5)
NEG = -1e30
HCHUNK = 8
NSPLIT = 4
_ROWS_Q = 4 * KV // NSPLIT


def _scores_kernel(qr_ref, ik_ref, s_ref):
    for j in range(4):
        qr = qr_ref[j]
        ik = ik_ref[j]
        s8 = lax.dot_general(qr, ik, (((1,), (1,)), ((), ())),
                             preferred_element_type=jnp.float32)
        s_ref[j] = jnp.sum(jnp.maximum(s8, 0.0), axis=0, keepdims=True)


def _thresh_kernel(s_ref, bt_ref):
    s = s_ref[:, 0, :]
    si = lax.bitcast_convert_type(s, jnp.int32)

    def body(_, carry):
        lo, hi = carry
        mid = lo + ((hi - lo) >> 1)
        ge = (si >= mid).astype(jnp.float32)
        cnt = jnp.sum(ge, axis=1, keepdims=True)
        pred = cnt >= TOPK
        return jnp.where(pred, mid, lo), jnp.where(pred, hi, mid)

    lo0 = jnp.zeros((B, 1), jnp.int32)
    hi0 = jnp.full((B, 1), 0x7F800000, jnp.int32)
    lo, _ = lax.fori_loop(0, 31, body, (lo0, hi0))
    m01 = (si >= lo).astype(jnp.float32)
    eye = (lax.broadcasted_iota(jnp.int32, (B, B), 0)
           == lax.broadcasted_iota(jnp.int32, (B, B), 1)).astype(jnp.float32)
    m01t = lax.dot_general(m01, eye, (((0,), (0,)), ((), ())),
                           preferred_element_type=jnp.float32)
    bt_ref[...] = NEG * (1.0 - m01t)


def _qabsorb_kernel(qn_ref, kbt_ref, o_ref):
    for i in range(HCHUNK):
        qn = qn_ref[:, i, :].astype(jnp.bfloat16)
        kbt = kbt_ref[i].astype(jnp.bfloat16)
        o_ref[:, i, :] = (SOFTMAX_SCALE * lax.dot_general(
            qn, kbt, (((1,), (1,)), ((), ())),
            preferred_element_type=jnp.float32)).astype(jnp.bfloat16)


def _attn_kernel(qno_ref, qr_ref, kv_hbm, bt_ref, o_ref,
                 buf0, buf1, sems0, sems1):
    b = pl.program_id(0)

    def issue(buf, sems, bb):
        for qi in range(NSPLIT):
            pltpu.make_async_copy(
                kv_hbm.at[pl.ds(bb * 4 * KV + qi * _ROWS_Q, _ROWS_Q), :],
                buf.at[pl.ds(qi * _ROWS_Q, _ROWS_Q), :],
                sems.at[qi],
            ).start()

    def wait(buf, sems, bb):
        for qi in range(NSPLIT):
            pltpu.make_async_copy(
                kv_hbm.at[pl.ds(bb * 4 * KV + qi * _ROWS_Q, _ROWS_Q), :],
                buf.at[pl.ds(qi * _ROWS_Q, _ROWS_Q), :],
                sems.at[qi],
            ).wait()

    @pl.when(b == 0)
    def _():
        issue(buf0, sems0, 0)

    @pl.when((b % 2 == 0) & (b + 1 < B // 4))
    def _():
        issue(buf1, sems1, b + 1)

    @pl.when((b % 2 == 1) & (b + 1 < B // 4))
    def _():
        issue(buf0, sems0, b + 1)

    def compute(buf, sems):
        wait(buf, sems, b)
        for j in range(4):
            qno = qno_ref[j]
            qrope = (qr_ref[j] * SOFTMAX_SCALE).astype(jnp.bfloat16)
            kv = buf[pl.ds(j * KV, KV), :].astype(jnp.bfloat16)
            onehot = (lax.broadcasted_iota(jnp.int32, (1, B), 1)
                      == 4 * b + j).astype(jnp.float32)
            bias = lax.dot_general(
                bt_ref[...], onehot, (((1,), (1,)), ((), ())),
                preferred_element_type=jnp.float32)
            lt = lax.dot_general(
                kv[:, :KV_LORA], qno, (((1,), (1,)), ((), ())),
                preferred_element_type=jnp.float32)
            lt += lax.dot_general(
                kv[:, KV_LORA:], qrope, (((1,), (1,)), ((), ())),
                preferred_element_type=jnp.float32)
            lt += bias
            m = jnp.max(lt, axis=0, keepdims=True)
            p = jnp.exp(lt - m)
            attn_t = (p / jnp.sum(p, axis=0,
                                  keepdims=True)).astype(jnp.bfloat16)
            o_ref[j] = lax.dot_general(
                attn_t, kv[:, :KV_LORA], (((0,), (0,)), ((), ())),
                preferred_element_type=jnp.float32).astype(jnp.bfloat16)

    @pl.when(b % 2 == 0)
    def _():
        compute(buf0, sems0)

    @pl.when(b % 2 == 1)
    def _():
        compute(buf1, sems1)


def _oproj_kernel(ao_ref, vb_ref, o_ref):
    for i in range(HCHUNK):
        ao = ao_ref[:, i, :]
        vb = vb_ref[i].astype(jnp.bfloat16)
        o_ref[:, i, :] = lax.dot_general(
            ao, vb, (((1,), (1,)), ((), ())),
            preferred_element_type=jnp.float32)


@jax.jit
def kernel(qr, q, indexer_k, latent_cache, k_b_proj_trans, v_b_proj):
    scores = pl.pallas_call(
        _scores_kernel,
        grid=(B // 4,),
        in_specs=[
            pl.BlockSpec((4, IDX_HEADS, IDX_DIM), lambda b: (b, 0, 0)),
            pl.BlockSpec((4, KV, IDX_DIM), lambda b: (b, 0, 0)),
        ],
        out_specs=pl.BlockSpec((4, 1, KV), lambda b: (b, 0, 0)),
        out_shape=jax.ShapeDtypeStruct((B, 1, KV), jnp.float32),
    )(qr, indexer_k)

    bias_t = pl.pallas_call(
        _thresh_kernel,
        out_shape=jax.ShapeDtypeStruct((KV, B), jnp.float32),
    )(scores)

    q_nope = q[..., :QK_NOPE]
    q_rope = q[..., QK_NOPE:]

    qno = pl.pallas_call(
        _qabsorb_kernel,
        grid=(NUM_HEADS // HCHUNK,),
        in_specs=[
            pl.BlockSpec((B, HCHUNK, QK_NOPE), lambda h: (0, h, 0)),
            pl.BlockSpec((HCHUNK, KV_LORA, QK_NOPE), lambda h: (h, 0, 0)),
        ],
        out_specs=pl.BlockSpec((B, HCHUNK, KV_LORA), lambda h: (0, h, 0)),
        out_shape=jax.ShapeDtypeStruct((B, NUM_HEADS, KV_LORA), jnp.bfloat16),
    )(q_nope, k_b_proj_trans)

    ao = pl.pallas_call(
        _attn_kernel,
        grid=(B // 4,),
        in_specs=[
            pl.BlockSpec((4, NUM_HEADS, KV_LORA), lambda b: (b, 0, 0)),
            pl.BlockSpec((4, NUM_HEADS, QK_ROPE), lambda b: (b, 0, 0)),
            pl.BlockSpec(memory_space=pl.ANY),
            pl.BlockSpec((KV, B), lambda b: (0, 0)),
        ],
        out_specs=pl.BlockSpec((4, NUM_HEADS, KV_LORA), lambda b: (b, 0, 0)),
        out_shape=jax.ShapeDtypeStruct((B, NUM_HEADS, KV_LORA), jnp.bfloat16),
        scratch_shapes=[
            pltpu.VMEM((4 * KV, KV_LORA + QK_ROPE), jnp.float32),
            pltpu.VMEM((4 * KV, KV_LORA + QK_ROPE), jnp.float32),
            pltpu.SemaphoreType.DMA((NSPLIT,)),
            pltpu.SemaphoreType.DMA((NSPLIT,)),
        ],
    )(qno, q_rope,
      latent_cache.reshape(B * KV, KV_LORA + QK_ROPE), bias_t)

    out = pl.pallas_call(
        _oproj_kernel,
        grid=(NUM_HEADS // HCHUNK,),
        in_specs=[
            pl.BlockSpec((B, HCHUNK, KV_LORA), lambda h: (0, h, 0)),
            pl.BlockSpec((HCHUNK, V_DIM, KV_LORA), lambda h: (h, 0, 0)),
        ],
        out_specs=pl.BlockSpec((B, HCHUNK, V_DIM), lambda h: (0, h, 0)),
        out_shape=jax.ShapeDtypeStruct((B, NUM_HEADS, V_DIM), jnp.float32),
    )(ao, v_b_proj)

    return out.reshape(B, NUM_HEADS * V_DIM)


# speedup vs baseline: 1.3189x; 1.0082x over previous
import jax
import jax.numpy as jnp
from jax import lax
from jax.experimental import pallas as pl
from jax.experimental.pallas import tpu as pltpu

NUM_HEADS = 128
QK_NOPE = 128
QK_ROPE = 64
KV_LORA = 512
V_DIM = 128
TOPK = 1024
IDX_HEADS = 8
IDX_DIM = 64
B = 64
KV = 2048
SOFTMAX_SCALE = (KV_LORA + QK_ROPE) ** (-0.5)
NEG = -1e30
HCHUNK = 16
NSPLIT = 4
_ROWS_Q = 4 * KV // NSPLIT


def _scores_kernel(qr_ref, ik_ref, s_ref):
    for j in range(4):
        qr = qr_ref[j]
        ik = ik_ref[j]
        s8 = lax.dot_general(qr, ik, (((1,), (1,)), ((), ())),
                             preferred_element_type=jnp.float32)
        s_ref[j] = jnp.sum(jnp.maximum(s8, 0.0), axis=0, keepdims=True)


def _thresh_kernel(s_ref, bt_ref):
    s = s_ref[:, 0, :]
    si = lax.bitcast_convert_type(s, jnp.int32)

    def body(_, carry):
        lo, hi = carry
        mid = lo + ((hi - lo) >> 1)
        ge = (si >= mid).astype(jnp.float32)
        cnt = jnp.sum(ge, axis=1, keepdims=True)
        pred = cnt >= TOPK
        return jnp.where(pred, mid, lo), jnp.where(pred, hi, mid)

    lo0 = jnp.zeros((B, 1), jnp.int32)
    hi0 = jnp.full((B, 1), 0x7F800000, jnp.int32)
    lo, _ = lax.fori_loop(0, 31, body, (lo0, hi0))
    m01 = (si >= lo).astype(jnp.float32)
    eye = (lax.broadcasted_iota(jnp.int32, (B, B), 0)
           == lax.broadcasted_iota(jnp.int32, (B, B), 1)).astype(jnp.float32)
    m01t = lax.dot_general(m01, eye, (((0,), (0,)), ((), ())),
                           preferred_element_type=jnp.float32)
    bt_ref[...] = NEG * (1.0 - m01t)


def _qabsorb_kernel(qn_ref, kbt_ref, o_ref):
    for i in range(HCHUNK):
        qn = qn_ref[:, i, :].astype(jnp.bfloat16)
        kbt = kbt_ref[i].astype(jnp.bfloat16)
        o_ref[:, i, :] = (SOFTMAX_SCALE * lax.dot_general(
            qn, kbt, (((1,), (1,)), ((), ())),
            preferred_element_type=jnp.float32)).astype(jnp.bfloat16)


def _attn_kernel(qno_ref, qr_ref, kv_hbm, bt_ref, o_ref,
                 buf0, buf1, sems0, sems1):
    b = pl.program_id(0)

    def issue(buf, sems, bb):
        for qi in range(NSPLIT):
            pltpu.make_async_copy(
                kv_hbm.at[pl.ds(bb * 4 * KV + qi * _ROWS_Q, _ROWS_Q), :],
                buf.at[pl.ds(qi * _ROWS_Q, _ROWS_Q), :],
                sems.at[qi],
            ).start()

    def wait(buf, sems, bb):
        for qi in range(NSPLIT):
            pltpu.make_async_copy(
                kv_hbm.at[pl.ds(bb * 4 * KV + qi * _ROWS_Q, _ROWS_Q), :],
                buf.at[pl.ds(qi * _ROWS_Q, _ROWS_Q), :],
                sems.at[qi],
            ).wait()

    @pl.when(b == 0)
    def _():
        issue(buf0, sems0, 0)

    @pl.when((b % 2 == 0) & (b + 1 < B // 4))
    def _():
        issue(buf1, sems1, b + 1)

    @pl.when((b % 2 == 1) & (b + 1 < B // 4))
    def _():
        issue(buf0, sems0, b + 1)

    def compute(buf, sems):
        wait(buf, sems, b)
        for j in range(4):
            qno = qno_ref[j]
            qrope = (qr_ref[j] * SOFTMAX_SCALE).astype(jnp.bfloat16)
            kv = buf[pl.ds(j * KV, KV), :].astype(jnp.bfloat16)
            onehot = (lax.broadcasted_iota(jnp.int32, (1, B), 1)
                      == 4 * b + j).astype(jnp.float32)
            bias = lax.dot_general(
                bt_ref[...], onehot, (((1,), (1,)), ((), ())),
                preferred_element_type=jnp.float32)
            lt = lax.dot_general(
                kv[:, :KV_LORA], qno, (((1,), (1,)), ((), ())),
                preferred_element_type=jnp.float32)
            lt += lax.dot_general(
                kv[:, KV_LORA:], qrope, (((1,), (1,)), ((), ())),
                preferred_element_type=jnp.float32)
            lt += bias
            m = jnp.max(lt, axis=0, keepdims=True)
            p = jnp.exp(lt - m)
            attn_t = (p / jnp.sum(p, axis=0,
                                  keepdims=True)).astype(jnp.bfloat16)
            o_ref[j] = lax.dot_general(
                attn_t, kv[:, :KV_LORA], (((0,), (0,)), ((), ())),
                preferred_element_type=jnp.float32).astype(jnp.bfloat16)

    @pl.when(b % 2 == 0)
    def _():
        compute(buf0, sems0)

    @pl.when(b % 2 == 1)
    def _():
        compute(buf1, sems1)


def _oproj_kernel(ao_ref, vb_ref, o_ref):
    for i in range(HCHUNK):
        ao = ao_ref[:, i, :]
        vb = vb_ref[i].astype(jnp.bfloat16)
        o_ref[:, i, :] = lax.dot_general(
            ao, vb, (((1,), (1,)), ((), ())),
            preferred_element_type=jnp.float32)


@jax.jit
def kernel(qr, q, indexer_k, latent_cache, k_b_proj_trans, v_b_proj):
    scores = pl.pallas_call(
        _scores_kernel,
        grid=(B // 4,),
        in_specs=[
            pl.BlockSpec((4, IDX_HEADS, IDX_DIM), lambda b: (b, 0, 0)),
            pl.BlockSpec((4, KV, IDX_DIM), lambda b: (b, 0, 0)),
        ],
        out_specs=pl.BlockSpec((4, 1, KV), lambda b: (b, 0, 0)),
        out_shape=jax.ShapeDtypeStruct((B, 1, KV), jnp.float32),
    )(qr, indexer_k)

    bias_t = pl.pallas_call(
        _thresh_kernel,
        out_shape=jax.ShapeDtypeStruct((KV, B), jnp.float32),
    )(scores)

    q_nope = q[..., :QK_NOPE]
    q_rope = q[..., QK_NOPE:]

    qno = pl.pallas_call(
        _qabsorb_kernel,
        grid=(NUM_HEADS // HCHUNK,),
        in_specs=[
            pl.BlockSpec((B, HCHUNK, QK_NOPE), lambda h: (0, h, 0)),
            pl.BlockSpec((HCHUNK, KV_LORA, QK_NOPE), lambda h: (h, 0, 0)),
        ],
        out_specs=pl.BlockSpec((B, HCHUNK, KV_LORA), lambda h: (0, h, 0)),
        out_shape=jax.ShapeDtypeStruct((B, NUM_HEADS, KV_LORA), jnp.bfloat16),
    )(q_nope, k_b_proj_trans)

    ao = pl.pallas_call(
        _attn_kernel,
        grid=(B // 4,),
        in_specs=[
            pl.BlockSpec((4, NUM_HEADS, KV_LORA), lambda b: (b, 0, 0)),
            pl.BlockSpec((4, NUM_HEADS, QK_ROPE), lambda b: (b, 0, 0)),
            pl.BlockSpec(memory_space=pl.ANY),
            pl.BlockSpec((KV, B), lambda b: (0, 0)),
        ],
        out_specs=pl.BlockSpec((4, NUM_HEADS, KV_LORA), lambda b: (b, 0, 0)),
        out_shape=jax.ShapeDtypeStruct((B, NUM_HEADS, KV_LORA), jnp.bfloat16),
        scratch_shapes=[
            pltpu.VMEM((4 * KV, KV_LORA + QK_ROPE), jnp.float32),
            pltpu.VMEM((4 * KV, KV_LORA + QK_ROPE), jnp.float32),
            pltpu.SemaphoreType.DMA((NSPLIT,)),
            pltpu.SemaphoreType.DMA((NSPLIT,)),
        ],
    )(qno, q_rope,
      latent_cache.reshape(B * KV, KV_LORA + QK_ROPE), bias_t)

    out = pl.pallas_call(
        _oproj_kernel,
        grid=(NUM_HEADS // HCHUNK,),
        in_specs=[
            pl.BlockSpec((B, HCHUNK, KV_LORA), lambda h: (0, h, 0)),
            pl.BlockSpec((HCHUNK, V_DIM, KV_LORA), lambda h: (h, 0, 0)),
        ],
        out_specs=pl.BlockSpec((B, HCHUNK, V_DIM), lambda h: (0, h, 0)),
        out_shape=jax.ShapeDtypeStruct((B, NUM_HEADS, V_DIM), jnp.float32),
    )(ao, v_b_proj)

    return out.reshape(B, NUM_HEADS * V_DIM)


# speedup vs baseline: 1.3427x; 1.0181x over previous
import jax
import jax.numpy as jnp
from jax import lax
from jax.experimental import pallas as pl
from jax.experimental.pallas import tpu as pltpu

NUM_HEADS = 128
QK_NOPE = 128
QK_ROPE = 64
KV_LORA = 512
V_DIM = 128
TOPK = 1024
IDX_HEADS = 8
IDX_DIM = 64
B = 64
KV = 2048
SOFTMAX_SCALE = (KV_LORA + QK_ROPE) ** (-0.5)
NEG = -1e30
HCHUNK = 16
NSPLIT = 8
_ROWS_Q = 4 * KV // NSPLIT


def _scores_kernel(qr_ref, ik_ref, s_ref):
    for j in range(8):
        qr = qr_ref[j]
        ik = ik_ref[j]
        s8 = lax.dot_general(qr, ik, (((1,), (1,)), ((), ())),
                             preferred_element_type=jnp.float32)
        s_ref[j] = jnp.sum(jnp.maximum(s8, 0.0), axis=0, keepdims=True)


def _thresh_kernel(s_ref, bt_ref):
    s = s_ref[:, 0, :]
    si = lax.bitcast_convert_type(s, jnp.int32)

    def body(_, carry):
        lo, hi = carry
        mid = lo + ((hi - lo) >> 1)
        ge = (si >= mid).astype(jnp.float32)
        cnt = jnp.sum(ge, axis=1, keepdims=True)
        pred = cnt >= TOPK
        return jnp.where(pred, mid, lo), jnp.where(pred, hi, mid)

    lo0 = jnp.zeros((B, 1), jnp.int32)
    hi0 = jnp.full((B, 1), 0x7F800000, jnp.int32)
    lo, _ = lax.fori_loop(0, 31, body, (lo0, hi0))
    m01 = (si >= lo).astype(jnp.float32)
    eye = (lax.broadcasted_iota(jnp.int32, (B, B), 0)
           == lax.broadcasted_iota(jnp.int32, (B, B), 1)).astype(jnp.float32)
    m01t = lax.dot_general(m01, eye, (((0,), (0,)), ((), ())),
                           preferred_element_type=jnp.float32)
    bt_ref[...] = NEG * (1.0 - m01t)


def _qabsorb_kernel(qn_ref, kbt_ref, o_ref):
    for i in range(HCHUNK):
        qn = qn_ref[:, i, :].astype(jnp.bfloat16)
        kbt = kbt_ref[i].astype(jnp.bfloat16)
        o_ref[:, i, :] = (SOFTMAX_SCALE * lax.dot_general(
            qn, kbt, (((1,), (1,)), ((), ())),
            preferred_element_type=jnp.float32)).astype(jnp.bfloat16)


def _attn_kernel(qno_ref, qr_ref, kv_hbm, bt_ref, o_ref,
                 buf0, buf1, sems0, sems1):
    b = pl.program_id(0)

    def issue(buf, sems, bb):
        for qi in range(NSPLIT):
            pltpu.make_async_copy(
                kv_hbm.at[pl.ds(bb * 4 * KV + qi * _ROWS_Q, _ROWS_Q), :],
                buf.at[pl.ds(qi * _ROWS_Q, _ROWS_Q), :],
                sems.at[qi],
            ).start()

    def wait(buf, sems, bb):
        for qi in range(NSPLIT):
            pltpu.make_async_copy(
                kv_hbm.at[pl.ds(bb * 4 * KV + qi * _ROWS_Q, _ROWS_Q), :],
                buf.at[pl.ds(qi * _ROWS_Q, _ROWS_Q), :],
                sems.at[qi],
            ).wait()

    @pl.when(b == 0)
    def _():
        issue(buf0, sems0, 0)

    @pl.when((b % 2 == 0) & (b + 1 < B // 4))
    def _():
        issue(buf1, sems1, b + 1)

    @pl.when((b % 2 == 1) & (b + 1 < B // 4))
    def _():
        issue(buf0, sems0, b + 1)

    def compute(buf, sems):
        wait(buf, sems, b)
        for j in range(4):
            qno = qno_ref[j]
            qrope = (qr_ref[j] * SOFTMAX_SCALE).astype(jnp.bfloat16)
            kv = buf[pl.ds(j * KV, KV), :].astype(jnp.bfloat16)
            onehot = (lax.broadcasted_iota(jnp.int32, (1, B), 1)
                      == 4 * b + j).astype(jnp.float32)
            bias = lax.dot_general(
                bt_ref[...], onehot, (((1,), (1,)), ((), ())),
                preferred_element_type=jnp.float32)
            lt = lax.dot_general(
                kv[:, :KV_LORA], qno, (((1,), (1,)), ((), ())),
                preferred_element_type=jnp.float32)
            lt += lax.dot_general(
                kv[:, KV_LORA:], qrope, (((1,), (1,)), ((), ())),
                preferred_element_type=jnp.float32)
            p = jnp.exp(lt + bias)
            attn_t = (p / jnp.sum(p, axis=0,
                                  keepdims=True)).astype(jnp.bfloat16)
            o_ref[j] = lax.dot_general(
                attn_t, kv[:, :KV_LORA], (((0,), (0,)), ((), ())),
                preferred_element_type=jnp.float32).astype(jnp.bfloat16)

    @pl.when(b % 2 == 0)
    def _():
        compute(buf0, sems0)

    @pl.when(b % 2 == 1)
    def _():
        compute(buf1, sems1)


def _oproj_kernel(ao_ref, vb_ref, o_ref):
    for i in range(HCHUNK):
        ao = ao_ref[:, i, :]
        vb = vb_ref[i].astype(jnp.bfloat16)
        o_ref[:, i, :] = lax.dot_general(
            ao, vb, (((1,), (1,)), ((), ())),
            preferred_element_type=jnp.float32)


@jax.jit
def kernel(qr, q, indexer_k, latent_cache, k_b_proj_trans, v_b_proj):
    scores = pl.pallas_call(
        _scores_kernel,
        grid=(B // 8,),
        in_specs=[
            pl.BlockSpec((8, IDX_HEADS, IDX_DIM), lambda b: (b, 0, 0)),
            pl.BlockSpec((8, KV, IDX_DIM), lambda b: (b, 0, 0)),
        ],
        out_specs=pl.BlockSpec((8, 1, KV), lambda b: (b, 0, 0)),
        out_shape=jax.ShapeDtypeStruct((B, 1, KV), jnp.float32),
    )(qr, indexer_k)

    bias_t = pl.pallas_call(
        _thresh_kernel,
        out_shape=jax.ShapeDtypeStruct((KV, B), jnp.float32),
    )(scores)

    q_nope = q[..., :QK_NOPE]
    q_rope = q[..., QK_NOPE:]

    qno = pl.pallas_call(
        _qabsorb_kernel,
        grid=(NUM_HEADS // HCHUNK,),
        in_specs=[
            pl.BlockSpec((B, HCHUNK, QK_NOPE), lambda h: (0, h, 0)),
            pl.BlockSpec((HCHUNK, KV_LORA, QK_NOPE), lambda h: (h, 0, 0)),
        ],
        out_specs=pl.BlockSpec((B, HCHUNK, KV_LORA), lambda h: (0, h, 0)),
        out_shape=jax.ShapeDtypeStruct((B, NUM_HEADS, KV_LORA), jnp.bfloat16),
    )(q_nope, k_b_proj_trans)

    ao = pl.pallas_call(
        _attn_kernel,
        grid=(B // 4,),
        in_specs=[
            pl.BlockSpec((4, NUM_HEADS, KV_LORA), lambda b: (b, 0, 0)),
            pl.BlockSpec((4, NUM_HEADS, QK_ROPE), lambda b: (b, 0, 0)),
            pl.BlockSpec(memory_space=pl.ANY),
            pl.BlockSpec((KV, B), lambda b: (0, 0)),
        ],
        out_specs=pl.BlockSpec((4, NUM_HEADS, KV_LORA), lambda b: (b, 0, 0)),
        out_shape=jax.ShapeDtypeStruct((B, NUM_HEADS, KV_LORA), jnp.bfloat16),
        scratch_shapes=[
            pltpu.VMEM((4 * KV, KV_LORA + QK_ROPE), jnp.float32),
            pltpu.VMEM((4 * KV, KV_LORA + QK_ROPE), jnp.float32),
            pltpu.SemaphoreType.DMA((NSPLIT,)),
            pltpu.SemaphoreType.DMA((NSPLIT,)),
        ],
    )(qno, q_rope,
      latent_cache.reshape(B * KV, KV_LORA + QK_ROPE), bias_t)

    out = pl.pallas_call(
        _oproj_kernel,
        grid=(NUM_HEADS // HCHUNK,),
        in_specs=[
            pl.BlockSpec((B, HCHUNK, KV_LORA), lambda h: (0, h, 0)),
            pl.BlockSpec((HCHUNK, V_DIM, KV_LORA), lambda h: (h, 0, 0)),
        ],
        out_specs=pl.BlockSpec((B, HCHUNK, V_DIM), lambda h: (0, h, 0)),
        out_shape=jax.ShapeDtypeStruct((B, NUM_HEADS, V_DIM), jnp.float32),
    )(ao, v_b_proj)

    return out.reshape(B, NUM_HEADS * V_DIM)


# speedup vs baseline: 1.3444x; 1.0012x over previous
import jax
import jax.numpy as jnp
from jax import lax
from jax.experimental import pallas as pl
from jax.experimental.pallas import tpu as pltpu

NUM_HEADS = 128
QK_NOPE = 128
QK_ROPE = 64
KV_LORA = 512
V_DIM = 128
TOPK = 1024
IDX_HEADS = 8
IDX_DIM = 64
B = 64
KV = 2048
SOFTMAX_SCALE = (KV_LORA + QK_ROPE) ** (-0.5)
NEG = -1e30
HCHUNK = 16
NSPLIT = 16
_ROWS_Q = 4 * KV // NSPLIT


def _scores_kernel(qr_ref, ik_ref, s_ref):
    for j in range(8):
        qr = qr_ref[j]
        ik = ik_ref[j]
        s8 = lax.dot_general(qr, ik, (((1,), (1,)), ((), ())),
                             preferred_element_type=jnp.float32)
        s_ref[j] = jnp.sum(jnp.maximum(s8, 0.0), axis=0, keepdims=True)


def _thresh_kernel(s_ref, bt_ref):
    s = s_ref[:, 0, :]
    si = lax.bitcast_convert_type(s, jnp.int32)

    def body(_, carry):
        lo, hi = carry
        mid = lo + ((hi - lo) >> 1)
        ge = (si >= mid).astype(jnp.float32)
        cnt = jnp.sum(ge, axis=1, keepdims=True)
        pred = cnt >= TOPK
        return jnp.where(pred, mid, lo), jnp.where(pred, hi, mid)

    lo0 = jnp.zeros((B, 1), jnp.int32)
    hi0 = jnp.full((B, 1), 0x7F800000, jnp.int32)
    lo, _ = lax.fori_loop(0, 31, body, (lo0, hi0))
    m01 = (si >= lo).astype(jnp.float32)
    eye = (lax.broadcasted_iota(jnp.int32, (B, B), 0)
           == lax.broadcasted_iota(jnp.int32, (B, B), 1)).astype(jnp.float32)
    m01t = lax.dot_general(m01, eye, (((0,), (0,)), ((), ())),
                           preferred_element_type=jnp.float32)
    bt_ref[...] = NEG * (1.0 - m01t)


def _qabsorb_kernel(qn_ref, kbt_ref, o_ref):
    for i in range(HCHUNK):
        qn = qn_ref[:, i, :].astype(jnp.bfloat16)
        kbt = kbt_ref[i].astype(jnp.bfloat16)
        o_ref[:, i, :] = (SOFTMAX_SCALE * lax.dot_general(
            qn, kbt, (((1,), (1,)), ((), ())),
            preferred_element_type=jnp.float32)).astype(jnp.bfloat16)


def _attn_kernel(qno_ref, qr_ref, kv_hbm, bt_ref, o_ref,
                 buf0, buf1, sems0, sems1):
    b = pl.program_id(0)

    def issue(buf, sems, bb):
        for qi in range(NSPLIT):
            pltpu.make_async_copy(
                kv_hbm.at[pl.ds(bb * 4 * KV + qi * _ROWS_Q, _ROWS_Q), :],
                buf.at[pl.ds(qi * _ROWS_Q, _ROWS_Q), :],
                sems.at[qi],
            ).start()

    def wait(buf, sems, bb):
        for qi in range(NSPLIT):
            pltpu.make_async_copy(
                kv_hbm.at[pl.ds(bb * 4 * KV + qi * _ROWS_Q, _ROWS_Q), :],
                buf.at[pl.ds(qi * _ROWS_Q, _ROWS_Q), :],
                sems.at[qi],
            ).wait()

    @pl.when(b == 0)
    def _():
        issue(buf0, sems0, 0)

    @pl.when((b % 2 == 0) & (b + 1 < B // 4))
    def _():
        issue(buf1, sems1, b + 1)

    @pl.when((b % 2 == 1) & (b + 1 < B // 4))
    def _():
        issue(buf0, sems0, b + 1)

    def compute(buf, sems):
        wait(buf, sems, b)
        for j in range(4):
            qno = qno_ref[j]
            qrope = (qr_ref[j] * SOFTMAX_SCALE).astype(jnp.bfloat16)
            kv = buf[pl.ds(j * KV, KV), :].astype(jnp.bfloat16)
            onehot = (lax.broadcasted_iota(jnp.int32, (1, B), 1)
                      == 4 * b + j).astype(jnp.float32)
            bias = lax.dot_general(
                bt_ref[...], onehot, (((1,), (1,)), ((), ())),
                preferred_element_type=jnp.float32)
            lt = lax.dot_general(
                kv[:, :KV_LORA], qno, (((1,), (1,)), ((), ())),
                preferred_element_type=jnp.float32)
            lt += lax.dot_general(
                kv[:, KV_LORA:], qrope, (((1,), (1,)), ((), ())),
                preferred_element_type=jnp.float32)
            p = jnp.exp(lt + bias)
            attn_t = (p / jnp.sum(p, axis=0,
                                  keepdims=True)).astype(jnp.bfloat16)
            o_ref[j] = lax.dot_general(
                attn_t, kv[:, :KV_LORA], (((0,), (0,)), ((), ())),
                preferred_element_type=jnp.float32).astype(jnp.bfloat16)

    @pl.when(b % 2 == 0)
    def _():
        compute(buf0, sems0)

    @pl.when(b % 2 == 1)
    def _():
        compute(buf1, sems1)


def _oproj_kernel(ao_ref, vb_ref, o_ref):
    for i in range(HCHUNK):
        ao = ao_ref[:, i, :]
        vb = vb_ref[i].astype(jnp.bfloat16)
        o_ref[:, i, :] = lax.dot_general(
            ao, vb, (((1,), (1,)), ((), ())),
            preferred_element_type=jnp.float32)


@jax.jit
def kernel(qr, q, indexer_k, latent_cache, k_b_proj_trans, v_b_proj):
    scores = pl.pallas_call(
        _scores_kernel,
        grid=(B // 8,),
        in_specs=[
            pl.BlockSpec((8, IDX_HEADS, IDX_DIM), lambda b: (b, 0, 0)),
            pl.BlockSpec((8, KV, IDX_DIM), lambda b: (b, 0, 0)),
        ],
        out_specs=pl.BlockSpec((8, 1, KV), lambda b: (b, 0, 0)),
        out_shape=jax.ShapeDtypeStruct((B, 1, KV), jnp.float32),
    )(qr, indexer_k)

    bias_t = pl.pallas_call(
        _thresh_kernel,
        out_shape=jax.ShapeDtypeStruct((KV, B), jnp.float32),
    )(scores)

    q_nope = q[..., :QK_NOPE]
    q_rope = q[..., QK_NOPE:]

    qno = pl.pallas_call(
        _qabsorb_kernel,
        grid=(NUM_HEADS // HCHUNK,),
        in_specs=[
            pl.BlockSpec((B, HCHUNK, QK_NOPE), lambda h: (0, h, 0)),
            pl.BlockSpec((HCHUNK, KV_LORA, QK_NOPE), lambda h: (h, 0, 0)),
        ],
        out_specs=pl.BlockSpec((B, HCHUNK, KV_LORA), lambda h: (0, h, 0)),
        out_shape=jax.ShapeDtypeStruct((B, NUM_HEADS, KV_LORA), jnp.bfloat16),
    )(q_nope, k_b_proj_trans)

    ao = pl.pallas_call(
        _attn_kernel,
        grid=(B // 4,),
        in_specs=[
            pl.BlockSpec((4, NUM_HEADS, KV_LORA), lambda b: (b, 0, 0)),
            pl.BlockSpec((4, NUM_HEADS, QK_ROPE), lambda b: (b, 0, 0)),
            pl.BlockSpec(memory_space=pl.ANY),
            pl.BlockSpec((KV, B), lambda b: (0, 0)),
        ],
        out_specs=pl.BlockSpec((4, NUM_HEADS, KV_LORA), lambda b: (b, 0, 0)),
        out_shape=jax.ShapeDtypeStruct((B, NUM_HEADS, KV_LORA), jnp.bfloat16),
        scratch_shapes=[
            pltpu.VMEM((4 * KV, KV_LORA + QK_ROPE), jnp.float32),
            pltpu.VMEM((4 * KV, KV_LORA + QK_ROPE), jnp.float32),
            pltpu.SemaphoreType.DMA((NSPLIT,)),
            pltpu.SemaphoreType.DMA((NSPLIT,)),
        ],
    )(qno, q_rope,
      latent_cache.reshape(B * KV, KV_LORA + QK_ROPE), bias_t)

    out = pl.pallas_call(
        _oproj_kernel,
        grid=(NUM_HEADS // HCHUNK,),
        in_specs=[
            pl.BlockSpec((B, HCHUNK, KV_LORA), lambda h: (0, h, 0)),
            pl.BlockSpec((HCHUNK, V_DIM, KV_LORA), lambda h: (h, 0, 0)),
        ],
        out_specs=pl.BlockSpec((B, HCHUNK, V_DIM), lambda h: (0, h, 0)),
        out_shape=jax.ShapeDtypeStruct((B, NUM_HEADS, V_DIM), jnp.float32),
    )(ao, v_b_proj)

    return out.reshape(B, NUM_HEADS * V_DIM)
